# Initial kernel scaffold; baseline (speedup 1.0000x reference)
import jax
import jax.numpy as jnp
from jax.experimental import pallas as pl


def kernel(x, selector_weights, expert_weights, compressor_w, compressed_net_w):
    raise NotImplementedError("write your pallas kernel here")


# speedup vs baseline: 1.3086x; 1.3086x over previous
import functools

import jax
import jax.numpy as jnp
from jax.experimental import pallas as pl
from jax.experimental.pallas import tpu as pltpu

IN_FEATURES = 4096
OUT_FEATURES = 4096
NUM_FRAGMENTS = 32
FRAGMENT_SIZE = IN_FEATURES // NUM_FRAGMENTS
COMPRESSED = 512
BATCH = 64

TILE_N = 512
GRID_N = OUT_FEATURES // TILE_N

_CONTRACT_LAST = (((1,), (1,)), ((), ()))


def _fused_kernel(x_ref, s_ref, e_ref, wc_ref, we_ref, wn_ref, out_ref,
                  wx_ref, comp_ref):
    @pl.when(pl.program_id(0) == 0)
    def _prologue():
        xv = x_ref[...]
        scores = jnp.dot(xv, s_ref[...], preferred_element_type=jnp.float32)
        m = jnp.max(scores, axis=1, keepdims=True)
        ex = jnp.exp(scores - m)
        probs = ex / jnp.sum(ex, axis=1, keepdims=True)
        pe = jnp.dot(probs, e_ref[...], preferred_element_type=jnp.float32)
        wx = xv * pe
        wx_ref[...] = wx
        comp_ref[...] = jax.lax.dot_general(
            xv - wx, wc_ref[...], _CONTRACT_LAST,
            preferred_element_type=jnp.float32)

    out_ref[...] = (
        jnp.dot(wx_ref[...], we_ref[...], preferred_element_type=jnp.float32)
        + jax.lax.dot_general(comp_ref[...], wn_ref[...], _CONTRACT_LAST,
                              preferred_element_type=jnp.float32))


@functools.partial(jax.jit, static_argnames=())
def kernel(x, selector_weights, expert_weights, compressor_w, compressed_net_w):
    eye = jnp.eye(NUM_FRAGMENTS, dtype=x.dtype)
    s_mat = jnp.einsum('fi,fg->fig', selector_weights, eye).reshape(
        IN_FEATURES, NUM_FRAGMENTS)
    e_mat = jnp.repeat(eye, FRAGMENT_SIZE, axis=1)
    we = expert_weights.reshape(IN_FEATURES, OUT_FEATURES)

    return pl.pallas_call(
        _fused_kernel,
        grid=(GRID_N,),
        in_specs=[
            pl.BlockSpec((BATCH, IN_FEATURES), lambda j: (0, 0)),
            pl.BlockSpec((IN_FEATURES, NUM_FRAGMENTS), lambda j: (0, 0)),
            pl.BlockSpec((NUM_FRAGMENTS, IN_FEATURES), lambda j: (0, 0)),
            pl.BlockSpec((COMPRESSED, IN_FEATURES), lambda j: (0, 0)),
            pl.BlockSpec((IN_FEATURES, TILE_N), lambda j: (0, j)),
            pl.BlockSpec((TILE_N, COMPRESSED), lambda j: (j, 0)),
        ],
        out_specs=pl.BlockSpec((BATCH, TILE_N), lambda j: (0, j)),
        out_shape=jax.ShapeDtypeStruct((BATCH, OUT_FEATURES), x.dtype),
        scratch_shapes=[
            pltpu.VMEM((BATCH, IN_FEATURES), jnp.float32),
            pltpu.VMEM((BATCH, COMPRESSED), jnp.float32),
        ],
    )(x, s_mat, e_mat, compressor_w, we, compressed_net_w)


# speedup vs baseline: 1.3600x; 1.0393x over previous
import functools

import jax
import jax.numpy as jnp
from jax.experimental import pallas as pl
from jax.experimental.pallas import tpu as pltpu

IN_FEATURES = 4096
OUT_FEATURES = 4096
NUM_FRAGMENTS = 32
FRAGMENT_SIZE = IN_FEATURES // NUM_FRAGMENTS
COMPRESSED = 512
BATCH = 64

TILE_K = 512
GRID_K = IN_FEATURES // TILE_K

_CONTRACT_LAST = (((1,), (1,)), ((), ()))


def _fused_kernel(x_ref, s_ref, e_ref, wc_ref, we_ref, wn_hbm_ref,
                  out_ref, probs_ref, comp_ref, wn_ref, wn_sem):
    k = pl.program_id(0)
    wn_copy = pltpu.make_async_copy(wn_hbm_ref, wn_ref, wn_sem)

    @pl.when(k == 0)
    def _prologue():
        wn_copy.start()
        xv = x_ref[...]
        scores = jnp.dot(xv, s_ref[...], preferred_element_type=jnp.float32)
        m = jnp.max(scores, axis=1, keepdims=True)
        ex = jnp.exp(scores - m)
        probs_ref[...] = ex / jnp.sum(ex, axis=1, keepdims=True)

    xk = x_ref[:, pl.ds(k * TILE_K, TILE_K)]
    pe = jnp.dot(probs_ref[...], e_ref[...], preferred_element_type=jnp.float32)
    wxk = xk * pe
    expert = jnp.dot(wxk, we_ref[...], preferred_element_type=jnp.float32)
    cpart = jax.lax.dot_general(xk - wxk, wc_ref[...], _CONTRACT_LAST,
                                preferred_element_type=jnp.float32)

    @pl.when(k == 0)
    def _init():
        out_ref[...] = expert
        comp_ref[...] = cpart

    @pl.when(k > 0)
    def _accum():
        out_ref[...] += expert
        comp_ref[...] += cpart

    @pl.when(k == GRID_K - 1)
    def _epilogue():
        wn_copy.wait()
        out_ref[...] += jax.lax.dot_general(
            comp_ref[...], wn_ref[...], _CONTRACT_LAST,
            preferred_element_type=jnp.float32)


@functools.partial(jax.jit, static_argnames=())
def kernel(x, selector_weights, expert_weights, compressor_w, compressed_net_w):
    eye = jnp.eye(NUM_FRAGMENTS, dtype=x.dtype)
    s_mat = jnp.einsum('fi,fg->fig', selector_weights, eye).reshape(
        IN_FEATURES, NUM_FRAGMENTS)
    e_mat = jnp.repeat(eye, FRAGMENT_SIZE, axis=1)
    we = expert_weights.reshape(IN_FEATURES, OUT_FEATURES)

    return pl.pallas_call(
        _fused_kernel,
        grid=(GRID_K,),
        in_specs=[
            pl.BlockSpec((BATCH, IN_FEATURES), lambda k: (0, 0)),
            pl.BlockSpec((IN_FEATURES, NUM_FRAGMENTS), lambda k: (0, 0)),
            pl.BlockSpec((NUM_FRAGMENTS, TILE_K), lambda k: (0, k)),
            pl.BlockSpec((COMPRESSED, TILE_K), lambda k: (0, k)),
            pl.BlockSpec((TILE_K, OUT_FEATURES), lambda k: (k, 0)),
            pl.BlockSpec(memory_space=pl.ANY),
        ],
        out_specs=pl.BlockSpec((BATCH, OUT_FEATURES), lambda k: (0, 0)),
        out_shape=jax.ShapeDtypeStruct((BATCH, OUT_FEATURES), x.dtype),
        scratch_shapes=[
            pltpu.VMEM((BATCH, NUM_FRAGMENTS), jnp.float32),
            pltpu.VMEM((BATCH, COMPRESSED), jnp.float32),
            pltpu.VMEM((OUT_FEATURES, COMPRESSED), jnp.float32),
            pltpu.SemaphoreType.DMA,
        ],
    )(x, s_mat, e_mat, compressor_w, we, compressed_net_w)


# speedup vs baseline: 1.6041x; 1.1795x over previous
import functools

import jax
import jax.numpy as jnp
from jax.experimental import pallas as pl
from jax.experimental.pallas import tpu as pltpu

IN_FEATURES = 4096
OUT_FEATURES = 4096
NUM_FRAGMENTS = 32
FRAGMENT_SIZE = IN_FEATURES // NUM_FRAGMENTS
COMPRESSED = 512
BATCH = 64

TILE_K = 512
GRID_K = IN_FEATURES // TILE_K

_CONTRACT_LAST = (((1,), (1,)), ((), ()))


def _frag_mask(k):
    col_frag = (k * TILE_K + jax.lax.broadcasted_iota(
        jnp.int32, (NUM_FRAGMENTS, TILE_K), 1)) // FRAGMENT_SIZE
    frag = jax.lax.broadcasted_iota(jnp.int32, (NUM_FRAGMENTS, TILE_K), 0)
    return (col_frag == frag).astype(jnp.bfloat16)


def _fused_kernel(x_ref, sw_ref, wc_ref, we_ref, wn_hbm_ref,
                  out_ref, probs_ref, comp_ref, wn_ref, wn_sem):
    k = pl.program_id(0)
    wn_copy = pltpu.make_async_copy(wn_hbm_ref, wn_ref, wn_sem)

    @pl.when(k == 0)
    def _prologue():
        wn_copy.start()
        xs = (x_ref[...] * sw_ref[...]).astype(jnp.bfloat16)
        masks = jnp.concatenate(
            [_frag_mask(i) for i in range(GRID_K)], axis=1)
        scores = jax.lax.dot_general(
            xs, masks, _CONTRACT_LAST, preferred_element_type=jnp.float32)
        m = jnp.max(scores, axis=1, keepdims=True)
        ex = jnp.exp(scores - m)
        probs_ref[...] = ex / jnp.sum(ex, axis=1, keepdims=True)

    xk = x_ref[:, pl.ds(k * TILE_K, TILE_K)]
    pe = jnp.dot(probs_ref[...].astype(jnp.bfloat16), _frag_mask(k),
                 preferred_element_type=jnp.float32)
    wxk = xk * pe
    expert = jnp.dot(wxk.astype(jnp.bfloat16),
                     we_ref[...].astype(jnp.bfloat16),
                     preferred_element_type=jnp.float32)
    cpart = jax.lax.dot_general((xk - wxk).astype(jnp.bfloat16),
                                wc_ref[...].astype(jnp.bfloat16),
                                _CONTRACT_LAST,
                                preferred_element_type=jnp.float32)

    @pl.when(k == 0)
    def _init():
        out_ref[...] = expert
        comp_ref[...] = cpart

    @pl.when(k > 0)
    def _accum():
        out_ref[...] += expert
        comp_ref[...] += cpart

    @pl.when(k == GRID_K - 1)
    def _epilogue():
        wn_copy.wait()
        out_ref[...] += jax.lax.dot_general(
            comp_ref[...].astype(jnp.bfloat16),
            wn_ref[...].astype(jnp.bfloat16),
            _CONTRACT_LAST, preferred_element_type=jnp.float32)


@functools.partial(jax.jit, static_argnames=())
def kernel(x, selector_weights, expert_weights, compressor_w, compressed_net_w):
    sw_row = selector_weights.reshape(1, IN_FEATURES)
    we = expert_weights.reshape(IN_FEATURES, OUT_FEATURES)

    return pl.pallas_call(
        _fused_kernel,
        grid=(GRID_K,),
        in_specs=[
            pl.BlockSpec((BATCH, IN_FEATURES), lambda k: (0, 0)),
            pl.BlockSpec((1, IN_FEATURES), lambda k: (0, 0)),
            pl.BlockSpec((COMPRESSED, TILE_K), lambda k: (0, k)),
            pl.BlockSpec((TILE_K, OUT_FEATURES), lambda k: (k, 0)),
            pl.BlockSpec(memory_space=pl.ANY),
        ],
        out_specs=pl.BlockSpec((BATCH, OUT_FEATURES), lambda k: (0, 0)),
        out_shape=jax.ShapeDtypeStruct((BATCH, OUT_FEATURES), x.dtype),
        scratch_shapes=[
            pltpu.VMEM((BATCH, NUM_FRAGMENTS), jnp.float32),
            pltpu.VMEM((BATCH, COMPRESSED), jnp.float32),
            pltpu.VMEM((OUT_FEATURES, COMPRESSED), jnp.float32),
            pltpu.SemaphoreType.DMA,
        ],
    )(x, sw_row, compressor_w, we, compressed_net_w)
